# baseline (speedup 1.0000x reference)
import functools

import jax
import jax.numpy as jnp
from jax import lax
from jax.experimental import pallas as pl
from jax.experimental.pallas import tpu as pltpu
from jax.experimental.pallas import tpu_sc as plsc

_N_NODES = 10000
_N_PAD = 10240
_E = 320000
_NC, _NS = 2, 16
_CHE = _E // (_NC * _NS)
_SLICE = _N_PAD // _NS

_mesh = plsc.VectorSubcoreMesh(core_axis_name="c", subcore_axis_name="s")


@functools.partial(
    pl.kernel,
    mesh=_mesh,
    out_type=jax.ShapeDtypeStruct((_NC, _N_PAD), jnp.float32),
    scratch_types=[
        pltpu.VMEM((2, _CHE), jnp.int32),
        pltpu.VMEM((_N_PAD,), jnp.float32),
        pltpu.VMEM((_N_PAD,), jnp.int32),
        pltpu.VMEM((_CHE,), jnp.float32),
        pltpu.VMEM((_SLICE,), jnp.float32),
        pltpu.VMEM_SHARED((_N_PAD,), jnp.float32),
        pltpu.VMEM_SHARED((_N_PAD,), jnp.float32),
    ],
    compiler_params=pltpu.CompilerParams(use_tc_tiling_on_sc=False,
                                         needs_layout_passes=False),
)
def _sc_edge_weights(edge_hbm, out_hbm, ev, hist_v, iota_v, r_v, sl_v,
                     deg_sh, c_sh):
    cid = lax.axis_index("c")
    sid = lax.axis_index("s")

    for i in range(_SLICE // 16):
        sl_v[pl.ds(i * 16, 16)] = jnp.zeros((16,), jnp.float32)
    pltpu.sync_copy(sl_v, deg_sh.at[pl.ds(sid * _SLICE, _SLICE)])
    pltpu.sync_copy(sl_v, c_sh.at[pl.ds(sid * _SLICE, _SLICE)])

    def _zf(i, carry):
        hist_v[pl.ds(i * 16, 16)] = jnp.zeros((16,), jnp.float32)
        iota_v[pl.ds(i * 16, 16)] = lax.iota(jnp.int32, 16) + i * 16
        return carry

    lax.fori_loop(0, _N_PAD // 16, _zf, 0)
    plsc.subcore_barrier()

    ones16 = jnp.full((16,), 1.0, jnp.float32)
    for r in range(2):
        base = (sid * 2 + r) * _CHE
        pltpu.sync_copy(edge_hbm.at[:, pl.ds(base, _CHE)], ev)

        def _hist(j, carry):
            for k in range(25):
                idx = ev[1, pl.ds(j * 400 + k * 16, 16)]
                plsc.addupdate_scatter(hist_v, [idx], ones16)
            return carry

        lax.fori_loop(0, _CHE // 400, _hist, 0)
    pltpu.sync_copy(hist_v, deg_sh.at[iota_v], add=True)
    plsc.subcore_barrier()

    nbase = sid * _SLICE
    pltpu.sync_copy(deg_sh.at[pl.ds(nbase, _SLICE)], sl_v)
    for i in range(_SLICE // 16):
        v = sl_v[pl.ds(i * 16, 16)]
        sl_v[pl.ds(i * 16, 16)] = 1.0 / jnp.maximum(v, 1.0)
    pltpu.sync_copy(sl_v, deg_sh.at[pl.ds(nbase, _SLICE)])
    plsc.subcore_barrier()

    wid = cid * _NS + sid
    pltpu.sync_copy(edge_hbm.at[:, pl.ds(wid * _CHE, _CHE)], ev)
    pltpu.sync_copy(deg_sh.at[ev.at[1]], r_v)
    pltpu.sync_copy(r_v, c_sh.at[ev.at[0]], add=True)
    plsc.subcore_barrier()

    @pl.when(sid == 0)
    def _flush():
        pltpu.sync_copy(c_sh, out_hbm.at[cid])


def _tc_mlp_body(x_ref, w1_ref, b1c_ref, w2t_ref, b2c_ref, w3t_ref, b3c_ref,
                 wct_ref, y_ref):
    dn = (((1,), (0,)), ((), ()))
    t1 = lax.dot_general(x_ref[...], w1_ref[...], dn,
                         preferred_element_type=jnp.float32)
    h = jnp.tanh(t1.T + b1c_ref[...])
    h = jnp.tanh(lax.dot_general(w2t_ref[...], h, dn,
                                 preferred_element_type=jnp.float32)
                 + b2c_ref[...])
    h = jnp.tanh(lax.dot_general(w3t_ref[...], h, dn,
                                 preferred_element_type=jnp.float32)
                 + b3c_ref[...])
    y_ref[...] = lax.dot_general(wct_ref[...], h, dn,
                                 preferred_element_type=jnp.float32)


def _tc_final_body(c_ref, y_ref, bc_ref, o_ref):
    c = c_ref[...]
    w = c[0:1, :_N_NODES] + c[1:2, :_N_NODES]
    z = (jnp.sum(w * y_ref[...], axis=1, keepdims=True) * (1.0 / _N_NODES)
         + bc_ref[...])
    o_ref[...] = 1.0 / (1.0 + jnp.exp(-z))


def kernel(x, edge_index, W1, b1, W2, b2, W3, b3, Wc, bc):
    cpart = _sc_edge_weights(edge_index)
    y = pl.pallas_call(
        _tc_mlp_body,
        out_shape=jax.ShapeDtypeStruct((1, _N_NODES), jnp.float32),
    )(x, W1, b1.reshape(-1, 1), W2.T, b2.reshape(-1, 1), W3.T,
      b3.reshape(-1, 1), Wc.T)
    out = pl.pallas_call(
        _tc_final_body,
        out_shape=jax.ShapeDtypeStruct((1, 1), jnp.float32),
    )(cpart, y, bc.reshape(1, 1))
    return out.reshape(1)


# speedup vs baseline: 1.1525x; 1.1525x over previous
import functools

import jax
import jax.numpy as jnp
from jax import lax
from jax.experimental import pallas as pl
from jax.experimental.pallas import tpu as pltpu
from jax.experimental.pallas import tpu_sc as plsc

_N_NODES = 10000
_N_PAD = 10240
_E = 320000
_NC, _NS = 2, 16
_CHE = _E // (_NC * _NS)
_SLICE = _N_PAD // _NS

_mesh = plsc.VectorSubcoreMesh(core_axis_name="c", subcore_axis_name="s")


@functools.partial(
    pl.kernel,
    mesh=_mesh,
    out_type=jax.ShapeDtypeStruct((_NC, _N_PAD), jnp.float32),
    scratch_types=[
        pltpu.VMEM((2, _CHE), jnp.int32),
        pltpu.VMEM((_CHE,), jnp.float32),
        pltpu.VMEM((_CHE,), jnp.float32),
        pltpu.VMEM((_SLICE,), jnp.float32),
        pltpu.VMEM_SHARED((_N_PAD,), jnp.float32),
        pltpu.VMEM_SHARED((_N_PAD,), jnp.float32),
    ],
    compiler_params=pltpu.CompilerParams(use_tc_tiling_on_sc=False),
)
def _sc_edge_weights(edge_hbm, out_hbm, ev, one_v, r_v, sl_v, deg_sh, c_sh):
    cid = lax.axis_index("c")
    sid = lax.axis_index("s")

    with jax.named_scope("p0_fill"):
        for i in range(_SLICE // 16):
            sl_v[pl.ds(i * 16, 16)] = jnp.zeros((16,), jnp.float32)
        pltpu.sync_copy(sl_v, deg_sh.at[pl.ds(sid * _SLICE, _SLICE)])
        pltpu.sync_copy(sl_v, c_sh.at[pl.ds(sid * _SLICE, _SLICE)])
        for i in range(_CHE // 16):
            one_v[pl.ds(i * 16, 16)] = jnp.full((16,), 1.0, jnp.float32)
    plsc.subcore_barrier()

    with jax.named_scope("p1_hist"):
        for r in range(2):
            base = (sid * 2 + r) * _CHE
            pltpu.sync_copy(edge_hbm.at[:, pl.ds(base, _CHE)], ev)
            pltpu.sync_copy(one_v, deg_sh.at[ev.at[1]], add=True)
    plsc.subcore_barrier()

    with jax.named_scope("p2_inv"):
        nbase = sid * _SLICE
        pltpu.sync_copy(deg_sh.at[pl.ds(nbase, _SLICE)], sl_v)
        for i in range(_SLICE // 16):
            v = sl_v[pl.ds(i * 16, 16)]
            sl_v[pl.ds(i * 16, 16)] = 1.0 / jnp.maximum(v, 1.0)
        pltpu.sync_copy(sl_v, deg_sh.at[pl.ds(nbase, _SLICE)])
    plsc.subcore_barrier()

    with jax.named_scope("p3_cw"):
        wid = cid * _NS + sid
        pltpu.sync_copy(edge_hbm.at[:, pl.ds(wid * _CHE, _CHE)], ev)
        pltpu.sync_copy(deg_sh.at[ev.at[1]], r_v)
        pltpu.sync_copy(r_v, c_sh.at[ev.at[0]], add=True)
    plsc.subcore_barrier()

    @pl.when(sid == 0)
    def _flush():
        with jax.named_scope("p4_flush"):
            pltpu.sync_copy(c_sh, out_hbm.at[cid])


def _tc_mlp_body(x_ref, w1_ref, b1c_ref, w2t_ref, b2c_ref, w3t_ref, b3c_ref,
                 wct_ref, y_ref):
    dn = (((1,), (0,)), ((), ()))
    t1 = lax.dot_general(x_ref[...], w1_ref[...], dn,
                         preferred_element_type=jnp.float32)
    h = jnp.tanh(t1.T + b1c_ref[...])
    h = jnp.tanh(lax.dot_general(w2t_ref[...], h, dn,
                                 preferred_element_type=jnp.float32)
                 + b2c_ref[...])
    h = jnp.tanh(lax.dot_general(w3t_ref[...], h, dn,
                                 preferred_element_type=jnp.float32)
                 + b3c_ref[...])
    y_ref[...] = lax.dot_general(wct_ref[...], h, dn,
                                 preferred_element_type=jnp.float32)


def _tc_final_body(c_ref, y_ref, bc_ref, o_ref):
    c = c_ref[...]
    w = c[0:1, :_N_NODES] + c[1:2, :_N_NODES]
    z = (jnp.sum(w * y_ref[...], axis=1, keepdims=True) * (1.0 / _N_NODES)
         + bc_ref[...])
    o_ref[...] = 1.0 / (1.0 + jnp.exp(-z))


def kernel(x, edge_index, W1, b1, W2, b2, W3, b3, Wc, bc):
    cpart = _sc_edge_weights(edge_index)
    y = pl.pallas_call(
        _tc_mlp_body,
        out_shape=jax.ShapeDtypeStruct((1, _N_NODES), jnp.float32),
    )(x, W1, b1.reshape(-1, 1), W2.T, b2.reshape(-1, 1), W3.T,
      b3.reshape(-1, 1), Wc.T)
    out = pl.pallas_call(
        _tc_final_body,
        out_shape=jax.ShapeDtypeStruct((1, 1), jnp.float32),
    )(cpart, y, bc.reshape(1, 1))
    return out.reshape(1)


# speedup vs baseline: 1.2043x; 1.0450x over previous
import functools

import jax
import jax.numpy as jnp
from jax import lax
from jax.experimental import pallas as pl
from jax.experimental.pallas import tpu as pltpu
from jax.experimental.pallas import tpu_sc as plsc

_N_NODES = 10000
_N_PAD = 10240
_E = 320000
_NC, _NS = 2, 16
_CHE = _E // (_NC * _NS)
_SLICE = _N_PAD // _NS

_mesh = plsc.VectorSubcoreMesh(core_axis_name="c", subcore_axis_name="s")


@functools.partial(
    pl.kernel,
    mesh=_mesh,
    out_type=jax.ShapeDtypeStruct((_NC, _N_PAD), jnp.float32),
    scratch_types=[
        pltpu.VMEM((2, _CHE), jnp.int32),
        pltpu.VMEM((2, _CHE), jnp.int32),
        pltpu.VMEM((2, _CHE), jnp.int32),
        pltpu.VMEM((_CHE,), jnp.float32),
        pltpu.VMEM((_CHE,), jnp.float32),
        pltpu.VMEM((_SLICE,), jnp.float32),
        pltpu.VMEM_SHARED((_N_PAD,), jnp.float32),
        pltpu.VMEM_SHARED((_N_PAD,), jnp.float32),
        pltpu.SemaphoreType.DMA,
        pltpu.SemaphoreType.DMA,
        pltpu.SemaphoreType.DMA,
    ],
    compiler_params=pltpu.CompilerParams(use_tc_tiling_on_sc=False),
)
def _sc_edge_weights(edge_hbm, out_hbm, ev0, ev1, ev3, one_v, r_v, sl_v,
                     deg_sh, c_sh, sem0, sem1, sem3):
    cid = lax.axis_index("c")
    sid = lax.axis_index("s")
    wid = cid * _NS + sid

    cp0 = pltpu.async_copy(
        edge_hbm.at[:, pl.ds((sid * 2) * _CHE, _CHE)], ev0, sem0)
    cp1 = pltpu.async_copy(
        edge_hbm.at[:, pl.ds((sid * 2 + 1) * _CHE, _CHE)], ev1, sem1)
    cp3 = pltpu.async_copy(
        edge_hbm.at[:, pl.ds(wid * _CHE, _CHE)], ev3, sem3)

    with jax.named_scope("p0_fill"):
        for i in range(_SLICE // 16):
            sl_v[pl.ds(i * 16, 16)] = jnp.zeros((16,), jnp.float32)
        pltpu.sync_copy(sl_v, deg_sh.at[pl.ds(sid * _SLICE, _SLICE)])
        pltpu.sync_copy(sl_v, c_sh.at[pl.ds(sid * _SLICE, _SLICE)])
        for i in range(_CHE // 16):
            one_v[pl.ds(i * 16, 16)] = jnp.full((16,), 1.0, jnp.float32)
    plsc.subcore_barrier()

    with jax.named_scope("p1_hist"):
        cp0.wait()
        pltpu.sync_copy(one_v, deg_sh.at[ev0.at[1]], add=True)
        cp1.wait()
        pltpu.sync_copy(one_v, deg_sh.at[ev1.at[1]], add=True)
    plsc.subcore_barrier()

    with jax.named_scope("p2_inv"):
        nbase = sid * _SLICE
        pltpu.sync_copy(deg_sh.at[pl.ds(nbase, _SLICE)], sl_v)
        for i in range(_SLICE // 16):
            v = sl_v[pl.ds(i * 16, 16)]
            sl_v[pl.ds(i * 16, 16)] = 1.0 / jnp.maximum(v, 1.0)
        pltpu.sync_copy(sl_v, deg_sh.at[pl.ds(nbase, _SLICE)])
    plsc.subcore_barrier()

    with jax.named_scope("p3_cw"):
        cp3.wait()
        pltpu.sync_copy(deg_sh.at[ev3.at[1]], r_v)
        pltpu.sync_copy(r_v, c_sh.at[ev3.at[0]], add=True)
    plsc.subcore_barrier()

    @pl.when(sid == 0)
    def _flush():
        with jax.named_scope("p4_flush"):
            pltpu.sync_copy(c_sh, out_hbm.at[cid])


def _tc_mlp_body(x_ref, w1_ref, b1c_ref, w2t_ref, b2c_ref, w3t_ref, b3c_ref,
                 wct_ref, y_ref):
    dn = (((1,), (0,)), ((), ()))
    t1 = lax.dot_general(x_ref[...], w1_ref[...], dn,
                         preferred_element_type=jnp.float32)
    h = jnp.tanh(t1.T + b1c_ref[...])
    h = jnp.tanh(lax.dot_general(w2t_ref[...], h, dn,
                                 preferred_element_type=jnp.float32)
                 + b2c_ref[...])
    h = jnp.tanh(lax.dot_general(w3t_ref[...], h, dn,
                                 preferred_element_type=jnp.float32)
                 + b3c_ref[...])
    y_ref[...] = lax.dot_general(wct_ref[...], h, dn,
                                 preferred_element_type=jnp.float32)


def _tc_final_body(c_ref, y_ref, bc_ref, o_ref):
    c = c_ref[...]
    w = c[0:1, :_N_NODES] + c[1:2, :_N_NODES]
    z = (jnp.sum(w * y_ref[...], axis=1, keepdims=True) * (1.0 / _N_NODES)
         + bc_ref[...])
    o_ref[...] = 1.0 / (1.0 + jnp.exp(-z))


def kernel(x, edge_index, W1, b1, W2, b2, W3, b3, Wc, bc):
    cpart = _sc_edge_weights(edge_index)
    y = pl.pallas_call(
        _tc_mlp_body,
        out_shape=jax.ShapeDtypeStruct((1, _N_NODES), jnp.float32),
    )(x, W1, b1.reshape(-1, 1), W2.T, b2.reshape(-1, 1), W3.T,
      b3.reshape(-1, 1), Wc.T)
    out = pl.pallas_call(
        _tc_final_body,
        out_shape=jax.ShapeDtypeStruct((1, 1), jnp.float32),
    )(cpart, y, bc.reshape(1, 1))
    return out.reshape(1)
